# baseline (speedup 1.0000x reference)
import functools
import jax
import jax.numpy as jnp
from jax import lax
from jax.experimental import pallas as pl
from jax.experimental.pallas import tpu as pltpu
from jax.experimental.pallas import tpu_sc as plsc

_info = plsc.get_sparse_core_info()
_NC, _NS = _info.num_cores, _info.num_subcores
_NW = _NC * _NS


def _make_sc_copy(seq_len, d_model):
    rows_per_w = seq_len // _NW
    mesh = plsc.VectorSubcoreMesh(core_axis_name="c", subcore_axis_name="s")

    @functools.partial(
        pl.kernel,
        mesh=mesh,
        out_type=jax.ShapeDtypeStruct((seq_len, d_model), jnp.float32),
    )
    def k(enc_hbm, out_hbm):
        wid = lax.axis_index("s") * _NC + lax.axis_index("c")
        base = wid * rows_per_w
        pltpu.sync_copy(enc_hbm.at[pl.ds(base, rows_per_w)],
                        out_hbm.at[pl.ds(base, rows_per_w)])

    return k


def kernel(x, encoding):
    seq_len = x.shape[0]
    d_model = encoding.shape[1]
    return _make_sc_copy(seq_len, d_model)(encoding[:seq_len, :])


# speedup vs baseline: 23.1474x; 23.1474x over previous
import functools
import jax
import jax.numpy as jnp
from jax import lax
from jax.experimental import pallas as pl
from jax.experimental.pallas import tpu as pltpu
from jax.experimental.pallas import tpu_sc as plsc

_info = plsc.get_sparse_core_info()
_NC, _NS = _info.num_cores, _info.num_subcores
_NW = _NC * _NS
_CH = 32


def _make_sc_copy(seq_len, d_model):
    rows_per_w = seq_len // _NW
    n_chunks = rows_per_w // _CH
    mesh = plsc.VectorSubcoreMesh(core_axis_name="c", subcore_axis_name="s")

    @functools.partial(
        pl.kernel,
        mesh=mesh,
        out_type=jax.ShapeDtypeStruct((seq_len, d_model), jnp.float32),
        scratch_types=[
            pltpu.VMEM((_CH, d_model), jnp.float32),
            pltpu.VMEM((_CH, d_model), jnp.float32),
            pltpu.SemaphoreType.DMA,
            pltpu.SemaphoreType.DMA,
            pltpu.SemaphoreType.DMA,
            pltpu.SemaphoreType.DMA,
        ],
    )
    def k(enc_hbm, out_hbm, buf0, buf1, isem0, isem1, osem0, osem1):
        wid = lax.axis_index("s") * _NC + lax.axis_index("c")
        base = wid * rows_per_w
        bufs = (buf0, buf1)
        isems = (isem0, isem1)
        osems = (osem0, osem1)
        st = [None, None]
        for i in range(n_chunks):
            b = i % 2
            if st[b] is not None:
                st[b].wait()
            ld = pltpu.async_copy(
                enc_hbm.at[pl.ds(base + i * _CH, _CH)], bufs[b], isems[b])
            ld.wait()
            st[b] = pltpu.async_copy(
                bufs[b], out_hbm.at[pl.ds(base + i * _CH, _CH)], osems[b])
        for d in st:
            if d is not None:
                d.wait()

    return k


def kernel(x, encoding):
    seq_len = x.shape[0]
    d_model = encoding.shape[1]
    return _make_sc_copy(seq_len, d_model)(encoding[:seq_len, :])


# speedup vs baseline: 82.2418x; 3.5530x over previous
import numpy as np
import jax
import jax.numpy as jnp
from jax.experimental import pallas as pl

_BLOCK = 1024
_FINE = 128


def _make_tables(seq_len, d_model, block, fine):
    n_blocks = seq_len // block
    n_mid = block // fine
    c = np.arange(d_model, dtype=np.float64)
    denom = np.power(10000.0, c / d_model)
    even = (np.arange(d_model) % 2 == 0)[None, :]

    alpha = (block * np.arange(n_blocks, dtype=np.float64))[:, None] / denom[None, :]
    a1 = np.where(even, np.sin(alpha), np.cos(alpha)).astype(np.float32)[:, None, :]
    a2 = np.where(even, np.cos(alpha), -np.sin(alpha)).astype(np.float32)[:, None, :]

    mu = (fine * np.arange(n_mid, dtype=np.float64))[:, None] / denom[None, :]
    m1 = np.cos(mu).astype(np.float32)
    m2 = np.sin(mu).astype(np.float32)

    beta = np.arange(fine, dtype=np.float64)[:, None] / denom[None, :]
    b1 = np.cos(beta).astype(np.float32)
    b2 = np.sin(beta).astype(np.float32)
    return a1, a2, m1, m2, b1, b2


def _gen_body(a1_ref, a2_ref, m1_ref, m2_ref, b1_ref, b2_ref, out_ref):
    a1 = a1_ref[0]
    a2 = a2_ref[0]
    b1 = b1_ref[...]
    b2 = b2_ref[...]
    n_mid = m1_ref.shape[0]
    fine = b1.shape[0]
    for m in range(n_mid):
        m1 = m1_ref[m][None, :]
        m2 = m2_ref[m][None, :]
        g1 = a1 * m1 + a2 * m2
        g2 = a2 * m1 - a1 * m2
        out_ref[m * fine:(m + 1) * fine, :] = g1 * b1 + g2 * b2


def kernel(x, encoding):
    seq_len = x.shape[0]
    d_model = encoding.shape[1]
    block = _BLOCK
    fine = _FINE
    n_blocks = seq_len // block
    n_mid = block // fine
    a1, a2, m1, m2, b1, b2 = _make_tables(seq_len, d_model, block, fine)
    return pl.pallas_call(
        _gen_body,
        grid=(n_blocks,),
        in_specs=[
            pl.BlockSpec((1, 1, d_model), lambda i: (i, 0, 0)),
            pl.BlockSpec((1, 1, d_model), lambda i: (i, 0, 0)),
            pl.BlockSpec((n_mid, d_model), lambda i: (0, 0)),
            pl.BlockSpec((n_mid, d_model), lambda i: (0, 0)),
            pl.BlockSpec((fine, d_model), lambda i: (0, 0)),
            pl.BlockSpec((fine, d_model), lambda i: (0, 0)),
        ],
        out_specs=pl.BlockSpec((block, d_model), lambda i: (i, 0)),
        out_shape=jax.ShapeDtypeStruct((seq_len, d_model), jnp.float32),
    )(a1, a2, m1, m2, b1, b2)


# speedup vs baseline: 83.0429x; 1.0097x over previous
import numpy as np
import jax
import jax.numpy as jnp
from jax.experimental import pallas as pl
from jax.experimental.pallas import tpu as pltpu

_BLOCK = 1024
_FINE = 128
_SEED = 16


def _make_tables(seq_len, d_model, block, fine, seed):
    n_blocks = seq_len // block
    n_mid = block // fine
    n_coarse = fine // seed
    c = np.arange(d_model, dtype=np.float64)
    denom = np.power(10000.0, c / d_model)
    even = (np.arange(d_model) % 2 == 0)[None, :]

    alpha = (block * np.arange(n_blocks, dtype=np.float64))[:, None] / denom[None, :]
    a1 = np.where(even, np.sin(alpha), np.cos(alpha)).astype(np.float32)[:, None, :]
    a2 = np.where(even, np.cos(alpha), -np.sin(alpha)).astype(np.float32)[:, None, :]

    mu = (fine * np.arange(n_mid, dtype=np.float64))[:, None] / denom[None, :]
    m1 = np.cos(mu).astype(np.float32)
    m2 = np.sin(mu).astype(np.float32)

    rho = np.arange(seed, dtype=np.float64)[:, None] / denom[None, :]
    sb1 = np.cos(rho).astype(np.float32)
    sb2 = np.sin(rho).astype(np.float32)

    sigma = (seed * np.arange(n_coarse, dtype=np.float64))[:, None] / denom[None, :]
    sc1 = np.cos(sigma).astype(np.float32)
    sc2 = np.sin(sigma).astype(np.float32)
    return a1, a2, m1, m2, sb1, sb2, sc1, sc2


def _gen_body(a1_ref, a2_ref, m1_ref, m2_ref, sb1_ref, sb2_ref, sc1_ref,
              sc2_ref, out_ref, b1_scr, b2_scr):
    seed = sb1_ref.shape[0]

    @pl.when(pl.program_id(0) == 0)
    def _expand_fine_tables():
        sb1 = sb1_ref[...]
        sb2 = sb2_ref[...]
        for s in range(sc1_ref.shape[0]):
            c1 = sc1_ref[s][None, :]
            c2 = sc2_ref[s][None, :]
            b1_scr[s * seed:(s + 1) * seed, :] = c1 * sb1 - c2 * sb2
            b2_scr[s * seed:(s + 1) * seed, :] = c2 * sb1 + c1 * sb2

    a1 = a1_ref[0]
    a2 = a2_ref[0]
    b1 = b1_scr[...]
    b2 = b2_scr[...]
    n_mid = m1_ref.shape[0]
    fine = b1.shape[0]
    for m in range(n_mid):
        m1 = m1_ref[m][None, :]
        m2 = m2_ref[m][None, :]
        g1 = a1 * m1 + a2 * m2
        g2 = a2 * m1 - a1 * m2
        out_ref[m * fine:(m + 1) * fine, :] = g1 * b1 + g2 * b2


def kernel(x, encoding):
    seq_len = x.shape[0]
    d_model = encoding.shape[1]
    block = _BLOCK
    fine = _FINE
    seed = _SEED
    n_blocks = seq_len // block
    n_mid = block // fine
    n_coarse = fine // seed
    tables = _make_tables(seq_len, d_model, block, fine, seed)
    return pl.pallas_call(
        _gen_body,
        grid=(n_blocks,),
        in_specs=[
            pl.BlockSpec((1, 1, d_model), lambda i: (i, 0, 0)),
            pl.BlockSpec((1, 1, d_model), lambda i: (i, 0, 0)),
            pl.BlockSpec((n_mid, d_model), lambda i: (0, 0)),
            pl.BlockSpec((n_mid, d_model), lambda i: (0, 0)),
            pl.BlockSpec((seed, d_model), lambda i: (0, 0)),
            pl.BlockSpec((seed, d_model), lambda i: (0, 0)),
            pl.BlockSpec((n_coarse, d_model), lambda i: (0, 0)),
            pl.BlockSpec((n_coarse, d_model), lambda i: (0, 0)),
        ],
        out_specs=pl.BlockSpec((block, d_model), lambda i: (i, 0)),
        out_shape=jax.ShapeDtypeStruct((seq_len, d_model), jnp.float32),
        scratch_shapes=[
            pltpu.VMEM((fine, d_model), jnp.float32),
            pltpu.VMEM((fine, d_model), jnp.float32),
        ],
    )(*tables)
